# baseline (speedup 1.0000x reference)
import functools

import jax
import jax.numpy as jnp
from jax import lax
from jax.experimental import pallas as pl
from jax.experimental.pallas import tpu as pltpu
from jax.experimental.pallas import tpu_sc as plsc

N_TOK = 2048
D_IN = 768
HIDDEN = 24576
TOPK = 32

LANES = 16
NWORK = 32
ROWS_PER_W = N_TOK // NWORK
NVEC = HIDDEN // LANES
CHUNK = 256
NCH = HIDDEN // CHUNK
NSUP = NCH // 16


R_B1 = 256
H_B1 = 2048


def _encode_body(x_ref, w_ref, b_ref, p_ref):
    acc = jax.lax.dot_general(
        x_ref[...], w_ref[...], dimension_numbers=(((1,), (1,)), ((), ())),
        preferred_element_type=jnp.float32)
    p_ref[...] = jnp.maximum(acc + b_ref[...], 0.0)


def _encode(x, W_enc, b_enc):
    grid = (HIDDEN // H_B1, N_TOK // R_B1)
    return pl.pallas_call(
        _encode_body,
        grid=grid,
        in_specs=[
            pl.BlockSpec((R_B1, D_IN), lambda h, r: (r, 0)),
            pl.BlockSpec((H_B1, D_IN), lambda h, r: (h, 0)),
            pl.BlockSpec((1, H_B1), lambda h, r: (0, h)),
        ],
        out_specs=pl.BlockSpec((R_B1, H_B1), lambda h, r: (r, h)),
        out_shape=jax.ShapeDtypeStruct((N_TOK, HIDDEN), jnp.float32),
    )(x, W_enc, b_enc.reshape(1, HIDDEN))


_GDN = jax.lax.GatherDimensionNumbers(
    offset_dims=(), collapsed_slice_dims=(0,), start_index_map=(0,))


def _splat(v, k):
    idx = jnp.full((LANES, 1), k, jnp.int32)
    return jax.lax.gather(v, idx, _GDN, (1,),
                          mode=jax.lax.GatherScatterMode.PROMISE_IN_BOUNDS)


def _shuf(v, idx):
    return jax.lax.gather(v, idx.reshape(LANES, 1), _GDN, (1,),
                          mode=jax.lax.GatherScatterMode.PROMISE_IN_BOUNDS)


def _allmax(v):
    lane = jax.lax.iota(jnp.int32, LANES)
    for s in (8, 4, 2, 1):
        v = jnp.maximum(v, _shuf(v, jnp.bitwise_xor(lane, s)))
    return v


def _allmin(v):
    lane = jax.lax.iota(jnp.int32, LANES)
    for s in (8, 4, 2, 1):
        v = jnp.minimum(v, _shuf(v, jnp.bitwise_xor(lane, s)))
    return v


def _scalar0(v):
    return jnp.squeeze(jax.lax.slice(v, (0,), (1,)))


def _tmax(vals):
    vals = list(vals)
    while len(vals) > 1:
        nxt = [jnp.maximum(vals[i], vals[i + 1])
               for i in range(0, len(vals) - 1, 2)]
        if len(vals) % 2:
            nxt.append(vals[-1])
        vals = nxt
    return vals[0]


def _tmin(vals):
    vals = list(vals)
    while len(vals) > 1:
        nxt = [jnp.minimum(vals[i], vals[i + 1])
               for i in range(0, len(vals) - 1, 2)]
        if len(vals) % 2:
            nxt.append(vals[-1])
        vals = nxt
    return vals[0]


def _sc_body(p_hbm, wdec_hbm, bdec_hbm, out_hbm,
             rowa_v, rowb_v, cm_v, scm_v, sella_v, selhb_v, wrows_v, acc_v,
             bdec_v, sema, semb, wsem, wsem2):
    wid = lax.axis_index("s") * 2 + lax.axis_index("c")
    lane = jnp.arange(LANES, dtype=jnp.int32)
    neg = jnp.float32(float("-inf"))

    pltpu.sync_copy(bdec_hbm, bdec_v)

    z = jnp.full((LANES,), neg, jnp.float32)
    zi = jnp.zeros((LANES,), jnp.int32)
    big = jnp.full((LANES,), 2**30, jnp.int32)

    def process(row_v, t):
        def p1(c2, _):
            for q in range(2):
                c = c2 * 2 + q
                m = _tmax([row_v[pl.ds(c * CHUNK + u * LANES, LANES)]
                           for u in range(CHUNK // LANES)])
                cm_v[pl.ds(c * LANES, LANES)] = m
            return 0
        lax.fori_loop(0, NCH // 2, p1, 0)

        for s in range(NSUP):
            m = _tmax([cm_v[pl.ds((s * 16 + u) * LANES, LANES)]
                       for u in range(16)])
            scm_v[pl.ds(s * LANES, LANES)] = m

        def one_k(k, carry):
            rv0, rv1, ri0, ri1 = carry
            sv = [scm_v[pl.ds(s * LANES, LANES)] for s in range(NSUP)]
            ms = _allmax(_tmax(sv))
            sstar = _scalar0(_allmin(_tmin(
                [jnp.where(sv[s] == ms, s, big) for s in range(NSUP)])))
            gbase = sstar * 16 * LANES
            cv = [cm_v[pl.ds(gbase + u * LANES, LANES)] for u in range(16)]
            cstar = sstar * 16 + _scalar0(_allmin(_tmin(
                [jnp.where(cv[u] == ms, u, big) for u in range(16)])))
            base = cstar * CHUNK

            vs = [row_v[pl.ds(base + u * LANES, LANES)]
                  for u in range(CHUNK // LANES)]
            pos = _allmin(_tmin(
                [jnp.where(vs[u] == ms, u * LANES + lane, big)
                 for u in range(CHUNK // LANES)]))

            masked = []
            for u in range(CHUNK // LANES):
                v = jnp.where((u * LANES + lane) == pos, neg, vs[u])
                row_v[pl.ds(base + u * LANES, LANES)] = v
                masked.append(v)
            nm = _tmax(masked)
            cm_v[pl.ds(cstar * LANES, LANES)] = nm
            rel = cstar - sstar * 16
            sm = _tmax([jnp.where(u == rel, nm, cv[u]) for u in range(16)])
            scm_v[pl.ds(sstar * LANES, LANES)] = sm

            mi = base + pos
            rv0 = jnp.where(lane == k, ms, rv0)
            ri0 = jnp.where(lane == k, mi, ri0)
            rv1 = jnp.where(lane == k - 16, ms, rv1)
            ri1 = jnp.where(lane == k - 16, mi, ri1)
            return rv0, rv1, ri0, ri1

        rv0, rv1, ri0, ri1 = lax.fori_loop(0, TOPK, one_k, (z, z, zi, zi))
        return rv0, rv1, ri0, ri1

    def decode(t, rv0, rv1, ri0, ri1):
        sella_v[...] = ri0
        selhb_v[...] = ri1
        cp1 = pltpu.async_copy(wdec_hbm.at[sella_v],
                               wrows_v.at[pl.ds(0, LANES)], wsem)
        cp2 = pltpu.async_copy(wdec_hbm.at[selhb_v],
                               wrows_v.at[pl.ds(LANES, LANES)], wsem2)
        ws_lo = [_splat(rv0, k) for k in range(LANES)]
        ws_hi = [_splat(rv1, k) for k in range(LANES)]
        cp1.wait()

        himask = jnp.full((LANES,), -65536, jnp.int32)

        def dj1(j, _):
            lo = bdec_v[pl.ds(j * 2 * LANES, LANES)]
            hi = bdec_v[pl.ds(j * 2 * LANES + LANES, LANES)]
            for k in range(LANES):
                w = wrows_v[k, pl.ds(j * LANES, LANES)]
                wlo = jax.lax.bitcast_convert_type(w << 16, jnp.float32)
                whi = jax.lax.bitcast_convert_type(w & himask, jnp.float32)
                lo = lo + ws_lo[k] * wlo
                hi = hi + ws_lo[k] * whi
            acc_v[pl.ds(j * 2 * LANES, LANES)] = lo
            acc_v[pl.ds(j * 2 * LANES + LANES, LANES)] = hi
            return 0
        lax.fori_loop(0, D_IN // (2 * LANES), dj1, 0)
        cp2.wait()

        def dj2(j, _):
            lo = acc_v[pl.ds(j * 2 * LANES, LANES)]
            hi = acc_v[pl.ds(j * 2 * LANES + LANES, LANES)]
            for k in range(LANES):
                w = wrows_v[LANES + k, pl.ds(j * LANES, LANES)]
                wlo = jax.lax.bitcast_convert_type(w << 16, jnp.float32)
                whi = jax.lax.bitcast_convert_type(w & himask, jnp.float32)
                lo = lo + ws_hi[k] * wlo
                hi = hi + ws_hi[k] * whi
            acc_v[pl.ds(j * 2 * LANES, LANES)] = lo
            acc_v[pl.ds(j * 2 * LANES + LANES, LANES)] = hi
            return 0
        lax.fori_loop(0, D_IN // (2 * LANES), dj2, 0)
        pltpu.sync_copy(acc_v, out_hbm.at[t])

    t0 = wid * ROWS_PER_W
    pltpu.async_copy(p_hbm.at[t0], rowa_v, sema)

    def two_rows(ii, _):
        ta = t0 + 2 * ii
        pltpu.make_async_copy(p_hbm.at[ta], rowa_v, sema).wait()
        pltpu.async_copy(p_hbm.at[ta + 1], rowb_v, semb)
        rv0, rv1, ri0, ri1 = process(rowa_v, ta)
        decode(ta, rv0, rv1, ri0, ri1)
        pltpu.make_async_copy(p_hbm.at[ta + 1], rowb_v, semb).wait()

        @pl.when(ii < ROWS_PER_W // 2 - 1)
        def _():
            pltpu.async_copy(p_hbm.at[ta + 2], rowa_v, sema)
        rv0b, rv1b, ri0b, ri1b = process(rowb_v, ta + 1)
        decode(ta + 1, rv0b, rv1b, ri0b, ri1b)
        return 0

    lax.fori_loop(0, ROWS_PER_W // 2, two_rows, 0)


def _sc_topk_decode(P, W_dec, b_dec):
    mesh = plsc.VectorSubcoreMesh(core_axis_name="c", subcore_axis_name="s")
    fn = pl.kernel(
        _sc_body, mesh=mesh,
        out_type=jax.ShapeDtypeStruct((N_TOK, D_IN), jnp.float32),
        scratch_types=[
            pltpu.VMEM((HIDDEN,), jnp.float32),
            pltpu.VMEM((HIDDEN,), jnp.float32),
            pltpu.VMEM((NCH * LANES,), jnp.float32),
            pltpu.VMEM((NSUP * LANES,), jnp.float32),
            pltpu.VMEM((LANES,), jnp.int32),
            pltpu.VMEM((LANES,), jnp.int32),
            pltpu.VMEM((TOPK, D_IN // 2), jnp.int32),
            pltpu.VMEM((D_IN,), jnp.float32),
            pltpu.VMEM((D_IN,), jnp.float32),
            pltpu.SemaphoreType.DMA,
            pltpu.SemaphoreType.DMA,
            pltpu.SemaphoreType.DMA,
            pltpu.SemaphoreType.DMA,
        ],
    )
    return fn(P, W_dec, b_dec)


@jax.jit
def kernel(x, W_enc, b_enc, W_dec, b_dec):
    sae_in = x - b_dec
    P = _encode(sae_in, W_enc, b_enc)
    wb = jax.lax.bitcast_convert_type(
        W_dec.astype(jnp.bfloat16), jnp.uint16).astype(jnp.uint32)
    wb = wb.reshape(HIDDEN, D_IN // 32, 2, LANES)
    words = (wb[:, :, 0, :] | (wb[:, :, 1, :] << 16)).astype(jnp.int32)
    W_pk = words.reshape(HIDDEN, D_IN // 2)
    return _sc_topk_decode(P, W_pk, b_dec)


# speedup vs baseline: 1.1440x; 1.1440x over previous
import functools

import jax
import jax.numpy as jnp
from jax import lax
from jax.experimental import pallas as pl
from jax.experimental.pallas import tpu as pltpu
from jax.experimental.pallas import tpu_sc as plsc

N_TOK = 2048
D_IN = 768
HIDDEN = 24576
TOPK = 32

LANES = 16
NWORK = 32
ROWS_PER_W = N_TOK // NWORK
NVEC = HIDDEN // LANES
CHUNK = 256
NCH = HIDDEN // CHUNK
NSUP = NCH // 16


R_B1 = 256
H_B1 = 2048


def _encode_body(x_ref, w_ref, b_ref, p_ref):
    acc = jax.lax.dot_general(
        x_ref[...], w_ref[...], dimension_numbers=(((1,), (1,)), ((), ())),
        preferred_element_type=jnp.float32)
    p_ref[...] = jnp.maximum(acc + b_ref[...], 0.0)


def _encode(x, W_enc, b_enc):
    grid = (HIDDEN // H_B1, N_TOK // R_B1)
    return pl.pallas_call(
        _encode_body,
        grid=grid,
        in_specs=[
            pl.BlockSpec((R_B1, D_IN), lambda h, r: (r, 0)),
            pl.BlockSpec((H_B1, D_IN), lambda h, r: (h, 0)),
            pl.BlockSpec((1, H_B1), lambda h, r: (0, h)),
        ],
        out_specs=pl.BlockSpec((R_B1, H_B1), lambda h, r: (r, h)),
        out_shape=jax.ShapeDtypeStruct((N_TOK, HIDDEN), jnp.float32),
    )(x, W_enc, b_enc.reshape(1, HIDDEN))


_GDN = jax.lax.GatherDimensionNumbers(
    offset_dims=(), collapsed_slice_dims=(0,), start_index_map=(0,))


def _splat(v, k):
    idx = jnp.full((LANES, 1), k, jnp.int32)
    return jax.lax.gather(v, idx, _GDN, (1,),
                          mode=jax.lax.GatherScatterMode.PROMISE_IN_BOUNDS)


def _shuf(v, idx):
    return jax.lax.gather(v, idx.reshape(LANES, 1), _GDN, (1,),
                          mode=jax.lax.GatherScatterMode.PROMISE_IN_BOUNDS)


def _allmax(v):
    lane = jax.lax.iota(jnp.int32, LANES)
    for s in (8, 4, 2, 1):
        v = jnp.maximum(v, _shuf(v, jnp.bitwise_xor(lane, s)))
    return v


def _allmin(v):
    lane = jax.lax.iota(jnp.int32, LANES)
    for s in (8, 4, 2, 1):
        v = jnp.minimum(v, _shuf(v, jnp.bitwise_xor(lane, s)))
    return v


def _scalar0(v):
    return jnp.squeeze(jax.lax.slice(v, (0,), (1,)))


def _tmax(vals):
    vals = list(vals)
    while len(vals) > 1:
        nxt = [jnp.maximum(vals[i], vals[i + 1])
               for i in range(0, len(vals) - 1, 2)]
        if len(vals) % 2:
            nxt.append(vals[-1])
        vals = nxt
    return vals[0]


def _tmin(vals):
    vals = list(vals)
    while len(vals) > 1:
        nxt = [jnp.minimum(vals[i], vals[i + 1])
               for i in range(0, len(vals) - 1, 2)]
        if len(vals) % 2:
            nxt.append(vals[-1])
        vals = nxt
    return vals[0]


def _sc_body(p_hbm, wdec_hbm, bdec_hbm, out_hbm,
             rowa_v, rowb_v, cm_v, scm_v, sella_v, selhb_v, wrows_v, acc_v,
             bdec_v, sema, semb, wsem, wsem2):
    wid = lax.axis_index("s") * 2 + lax.axis_index("c")
    lane = jnp.arange(LANES, dtype=jnp.int32)
    neg = jnp.float32(float("-inf"))

    pltpu.sync_copy(bdec_hbm, bdec_v)

    z = jnp.full((LANES,), neg, jnp.float32)
    zi = jnp.zeros((LANES,), jnp.int32)
    big = jnp.full((LANES,), 2**30, jnp.int32)

    def process(row_v, t):
        def p1(c2, _):
            for q in range(2):
                c = c2 * 2 + q
                m = _tmax([row_v[pl.ds(c * CHUNK + u * LANES, LANES)]
                           for u in range(CHUNK // LANES)])
                cm_v[pl.ds(c * LANES, LANES)] = m
            return 0
        lax.fori_loop(0, NCH // 2, p1, 0)

        for s in range(NSUP):
            m = _tmax([cm_v[pl.ds((s * 16 + u) * LANES, LANES)]
                       for u in range(16)])
            scm_v[pl.ds(s * LANES, LANES)] = m

        def one_k(k, carry):
            rv0, rv1, ri0, ri1 = carry
            sv = [scm_v[pl.ds(s * LANES, LANES)] for s in range(NSUP)]
            ms = _allmax(_tmax(sv))
            sstar = _scalar0(_allmin(_tmin(
                [jnp.where(sv[s] == ms, s, big) for s in range(NSUP)])))
            gbase = sstar * 16 * LANES
            cv = [cm_v[pl.ds(gbase + u * LANES, LANES)] for u in range(16)]
            cstar = sstar * 16 + _scalar0(_allmin(_tmin(
                [jnp.where(cv[u] == ms, u, big) for u in range(16)])))
            base = cstar * CHUNK

            vs = [row_v[pl.ds(base + u * LANES, LANES)]
                  for u in range(CHUNK // LANES)]
            pos = _allmin(_tmin(
                [jnp.where(vs[u] == ms, u * LANES + lane, big)
                 for u in range(CHUNK // LANES)]))

            masked = []
            for u in range(CHUNK // LANES):
                v = jnp.where((u * LANES + lane) == pos, neg, vs[u])
                row_v[pl.ds(base + u * LANES, LANES)] = v
                masked.append(v)
            nm = _tmax(masked)
            cm_v[pl.ds(cstar * LANES, LANES)] = nm
            rel = cstar - sstar * 16
            sm = _tmax([jnp.where(u == rel, nm, cv[u]) for u in range(16)])
            scm_v[pl.ds(sstar * LANES, LANES)] = sm

            mi = base + pos
            rv0 = jnp.where(lane == k, ms, rv0)
            ri0 = jnp.where(lane == k, mi, ri0)
            rv1 = jnp.where(lane == k - 16, ms, rv1)
            ri1 = jnp.where(lane == k - 16, mi, ri1)
            return rv0, rv1, ri0, ri1

        rv0, rv1, ri0, ri1 = lax.fori_loop(0, TOPK, one_k, (z, z, zi, zi))
        return rv0, rv1, ri0, ri1

    def decode_issue(ri0, ri1):
        sella_v[...] = ri0
        selhb_v[...] = ri1
        cp1 = pltpu.async_copy(wdec_hbm.at[sella_v],
                               wrows_v.at[pl.ds(0, LANES)], wsem)
        cp2 = pltpu.async_copy(wdec_hbm.at[selhb_v],
                               wrows_v.at[pl.ds(LANES, LANES)], wsem2)
        return cp1, cp2

    def decode_finish(t, rv0, rv1, cp1, cp2):
        ws_lo = [_splat(rv0, k) for k in range(LANES)]
        ws_hi = [_splat(rv1, k) for k in range(LANES)]
        cp1.wait()

        def dj1(j, _):
            sl = pl.ds(j * LANES, LANES)
            a = bdec_v[sl]
            for k in range(LANES):
                a = a + ws_lo[k] * wrows_v[k, sl]
            acc_v[sl] = a
            return 0
        lax.fori_loop(0, D_IN // LANES, dj1, 0)
        cp2.wait()

        def dj2(j, _):
            sl = pl.ds(j * LANES, LANES)
            a = acc_v[sl]
            for k in range(LANES):
                a = a + ws_hi[k] * wrows_v[LANES + k, sl]
            acc_v[sl] = a
            return 0
        lax.fori_loop(0, D_IN // LANES, dj2, 0)
        pltpu.sync_copy(acc_v, out_hbm.at[t])

    t0 = wid * ROWS_PER_W
    pltpu.async_copy(p_hbm.at[t0], rowa_v, sema)

    def two_rows(ii, _):
        ta = t0 + 2 * ii
        pltpu.make_async_copy(p_hbm.at[ta], rowa_v, sema).wait()
        pltpu.async_copy(p_hbm.at[ta + 1], rowb_v, semb)
        rv0, rv1, ri0, ri1 = process(rowa_v, ta)
        cpa1, cpa2 = decode_issue(ri0, ri1)
        pltpu.make_async_copy(p_hbm.at[ta + 1], rowb_v, semb).wait()

        @pl.when(ii < ROWS_PER_W // 2 - 1)
        def _():
            pltpu.async_copy(p_hbm.at[ta + 2], rowa_v, sema)
        rv0b, rv1b, ri0b, ri1b = process(rowb_v, ta + 1)
        decode_finish(ta, rv0, rv1, cpa1, cpa2)
        cpb1, cpb2 = decode_issue(ri0b, ri1b)
        decode_finish(ta + 1, rv0b, rv1b, cpb1, cpb2)
        return 0

    lax.fori_loop(0, ROWS_PER_W // 2, two_rows, 0)


def _sc_topk_decode(P, W_dec, b_dec):
    mesh = plsc.VectorSubcoreMesh(core_axis_name="c", subcore_axis_name="s")
    fn = pl.kernel(
        _sc_body, mesh=mesh,
        out_type=jax.ShapeDtypeStruct((N_TOK, D_IN), jnp.float32),
        scratch_types=[
            pltpu.VMEM((HIDDEN,), jnp.float32),
            pltpu.VMEM((HIDDEN,), jnp.float32),
            pltpu.VMEM((NCH * LANES,), jnp.float32),
            pltpu.VMEM((NSUP * LANES,), jnp.float32),
            pltpu.VMEM((LANES,), jnp.int32),
            pltpu.VMEM((LANES,), jnp.int32),
            pltpu.VMEM((TOPK, D_IN), jnp.float32),
            pltpu.VMEM((D_IN,), jnp.float32),
            pltpu.VMEM((D_IN,), jnp.float32),
            pltpu.SemaphoreType.DMA,
            pltpu.SemaphoreType.DMA,
            pltpu.SemaphoreType.DMA,
            pltpu.SemaphoreType.DMA,
        ],
    )
    return fn(P, W_dec, b_dec)


@jax.jit
def kernel(x, W_enc, b_enc, W_dec, b_dec):
    sae_in = x - b_dec
    P = _encode(sae_in, W_enc, b_enc)
    return _sc_topk_decode(P, W_dec, b_dec)


# speedup vs baseline: 1.2263x; 1.0719x over previous
import functools

import jax
import jax.numpy as jnp
from jax import lax
from jax.experimental import pallas as pl
from jax.experimental.pallas import tpu as pltpu
from jax.experimental.pallas import tpu_sc as plsc

N_TOK = 2048
D_IN = 768
HIDDEN = 24576
TOPK = 32

LANES = 16
NWORK = 32
ROWS_PER_W = N_TOK // NWORK
NVEC = HIDDEN // LANES
CHUNK = 256
NCH = HIDDEN // CHUNK
NSUP = NCH // 16


R_B1 = 256
H_B1 = 2048


def _encode_body(x_ref, w_ref, b_ref, p_ref):
    acc = jax.lax.dot_general(
        x_ref[...], w_ref[...], dimension_numbers=(((1,), (1,)), ((), ())),
        preferred_element_type=jnp.float32)
    p_ref[...] = jnp.maximum(acc + b_ref[...], 0.0)


def _encode(x, W_enc, b_enc):
    ntok = x.shape[0]
    grid = (HIDDEN // H_B1, ntok // R_B1)
    return pl.pallas_call(
        _encode_body,
        grid=grid,
        in_specs=[
            pl.BlockSpec((R_B1, D_IN), lambda h, r: (r, 0)),
            pl.BlockSpec((H_B1, D_IN), lambda h, r: (h, 0)),
            pl.BlockSpec((1, H_B1), lambda h, r: (0, h)),
        ],
        out_specs=pl.BlockSpec((R_B1, H_B1), lambda h, r: (r, h)),
        out_shape=jax.ShapeDtypeStruct((ntok, HIDDEN), jnp.float32),
    )(x, W_enc, b_enc.reshape(1, HIDDEN))


_GDN = jax.lax.GatherDimensionNumbers(
    offset_dims=(), collapsed_slice_dims=(0,), start_index_map=(0,))


def _splat(v, k):
    idx = jnp.full((LANES, 1), k, jnp.int32)
    return jax.lax.gather(v, idx, _GDN, (1,),
                          mode=jax.lax.GatherScatterMode.PROMISE_IN_BOUNDS)


def _shuf(v, idx):
    return jax.lax.gather(v, idx.reshape(LANES, 1), _GDN, (1,),
                          mode=jax.lax.GatherScatterMode.PROMISE_IN_BOUNDS)


def _allmax(v):
    lane = jax.lax.iota(jnp.int32, LANES)
    for s in (8, 4, 2, 1):
        v = jnp.maximum(v, _shuf(v, jnp.bitwise_xor(lane, s)))
    return v


def _allmin(v):
    lane = jax.lax.iota(jnp.int32, LANES)
    for s in (8, 4, 2, 1):
        v = jnp.minimum(v, _shuf(v, jnp.bitwise_xor(lane, s)))
    return v


def _scalar0(v):
    return jnp.squeeze(jax.lax.slice(v, (0,), (1,)))


def _tmax(vals):
    vals = list(vals)
    while len(vals) > 1:
        nxt = [jnp.maximum(vals[i], vals[i + 1])
               for i in range(0, len(vals) - 1, 2)]
        if len(vals) % 2:
            nxt.append(vals[-1])
        vals = nxt
    return vals[0]


def _tmin(vals):
    vals = list(vals)
    while len(vals) > 1:
        nxt = [jnp.minimum(vals[i], vals[i + 1])
               for i in range(0, len(vals) - 1, 2)]
        if len(vals) % 2:
            nxt.append(vals[-1])
        vals = nxt
    return vals[0]


def _sc_body(rows_per_w, p_hbm, wdec_hbm, bdec_hbm, out_hbm,
             rowa_v, rowb_v, cm_v, scm_v, sella_v, selhb_v, wrows_v, acc_v,
             bdec_v, sema, semb, wsem, wsem2):
    wid = lax.axis_index("s") * 2 + lax.axis_index("c")
    lane = jnp.arange(LANES, dtype=jnp.int32)
    neg = jnp.float32(float("-inf"))

    pltpu.sync_copy(bdec_hbm, bdec_v)

    z = jnp.full((LANES,), neg, jnp.float32)
    zi = jnp.zeros((LANES,), jnp.int32)
    big = jnp.full((LANES,), 2**30, jnp.int32)

    def process(row_v, t):
        def p1(c2, _):
            for q in range(2):
                c = c2 * 2 + q
                m = _tmax([row_v[pl.ds(c * CHUNK + u * LANES, LANES)]
                           for u in range(CHUNK // LANES)])
                cm_v[pl.ds(c * LANES, LANES)] = m
            return 0
        lax.fori_loop(0, NCH // 2, p1, 0)

        for s in range(NSUP):
            m = _tmax([cm_v[pl.ds((s * 16 + u) * LANES, LANES)]
                       for u in range(16)])
            scm_v[pl.ds(s * LANES, LANES)] = m

        def one_k(k, carry):
            rv0, rv1, ri0, ri1 = carry
            sv = [scm_v[pl.ds(s * LANES, LANES)] for s in range(NSUP)]
            ms = _allmax(_tmax(sv))
            sstar = _scalar0(_allmin(_tmin(
                [jnp.where(sv[s] == ms, s, big) for s in range(NSUP)])))
            gbase = sstar * 16 * LANES
            cv = [cm_v[pl.ds(gbase + u * LANES, LANES)] for u in range(16)]
            cstar = sstar * 16 + _scalar0(_allmin(_tmin(
                [jnp.where(cv[u] == ms, u, big) for u in range(16)])))
            base = cstar * CHUNK

            vs = [row_v[pl.ds(base + u * LANES, LANES)]
                  for u in range(CHUNK // LANES)]
            pos = _allmin(_tmin(
                [jnp.where(vs[u] == ms, u * LANES + lane, big)
                 for u in range(CHUNK // LANES)]))

            masked = []
            for u in range(CHUNK // LANES):
                v = jnp.where((u * LANES + lane) == pos, neg, vs[u])
                row_v[pl.ds(base + u * LANES, LANES)] = v
                masked.append(v)
            nm = _tmax(masked)
            cm_v[pl.ds(cstar * LANES, LANES)] = nm
            rel = cstar - sstar * 16
            sm = _tmax([jnp.where(u == rel, nm, cv[u]) for u in range(16)])
            scm_v[pl.ds(sstar * LANES, LANES)] = sm

            mi = base + pos
            rv0 = jnp.where(lane == k, ms, rv0)
            ri0 = jnp.where(lane == k, mi, ri0)
            rv1 = jnp.where(lane == k - 16, ms, rv1)
            ri1 = jnp.where(lane == k - 16, mi, ri1)
            return rv0, rv1, ri0, ri1

        rv0, rv1, ri0, ri1 = lax.fori_loop(0, TOPK, one_k, (z, z, zi, zi))
        return rv0, rv1, ri0, ri1

    def decode_issue(ri0, ri1):
        sella_v[...] = ri0
        selhb_v[...] = ri1
        cp1 = pltpu.async_copy(wdec_hbm.at[sella_v],
                               wrows_v.at[pl.ds(0, LANES)], wsem)
        cp2 = pltpu.async_copy(wdec_hbm.at[selhb_v],
                               wrows_v.at[pl.ds(LANES, LANES)], wsem2)
        return cp1, cp2

    def decode_finish(t, rv0, rv1, cp1, cp2):
        ws_lo = [_splat(rv0, k) for k in range(LANES)]
        ws_hi = [_splat(rv1, k) for k in range(LANES)]
        cp1.wait()

        def dj1(j, _):
            sl = pl.ds(j * LANES, LANES)
            a = bdec_v[sl]
            for k in range(LANES):
                a = a + ws_lo[k] * wrows_v[k, sl]
            acc_v[sl] = a
            return 0
        lax.fori_loop(0, D_IN // LANES, dj1, 0)
        cp2.wait()

        def dj2(j, _):
            sl = pl.ds(j * LANES, LANES)
            a = acc_v[sl]
            for k in range(LANES):
                a = a + ws_hi[k] * wrows_v[LANES + k, sl]
            acc_v[sl] = a
            return 0
        lax.fori_loop(0, D_IN // LANES, dj2, 0)
        pltpu.sync_copy(acc_v, out_hbm.at[t])

    t0 = wid * rows_per_w
    pltpu.async_copy(p_hbm.at[t0], rowa_v, sema)

    def two_rows(ii, _):
        ta = t0 + 2 * ii
        pltpu.make_async_copy(p_hbm.at[ta], rowa_v, sema).wait()
        pltpu.async_copy(p_hbm.at[ta + 1], rowb_v, semb)
        rv0, rv1, ri0, ri1 = process(rowa_v, ta)
        cpa1, cpa2 = decode_issue(ri0, ri1)
        pltpu.make_async_copy(p_hbm.at[ta + 1], rowb_v, semb).wait()

        @pl.when(ii < rows_per_w // 2 - 1)
        def _():
            pltpu.async_copy(p_hbm.at[ta + 2], rowa_v, sema)
        rv0b, rv1b, ri0b, ri1b = process(rowb_v, ta + 1)
        decode_finish(ta, rv0, rv1, cpa1, cpa2)
        cpb1, cpb2 = decode_issue(ri0b, ri1b)
        decode_finish(ta + 1, rv0b, rv1b, cpb1, cpb2)
        return 0

    lax.fori_loop(0, rows_per_w // 2, two_rows, 0)


def _sc_topk_decode(P, W_dec, b_dec):
    ntok = P.shape[0]
    mesh = plsc.VectorSubcoreMesh(core_axis_name="c", subcore_axis_name="s")
    fn = pl.kernel(
        functools.partial(_sc_body, ntok // NWORK), mesh=mesh,
        out_type=jax.ShapeDtypeStruct((ntok, D_IN), jnp.float32),
        scratch_types=[
            pltpu.VMEM((HIDDEN,), jnp.float32),
            pltpu.VMEM((HIDDEN,), jnp.float32),
            pltpu.VMEM((NCH * LANES,), jnp.float32),
            pltpu.VMEM((NSUP * LANES,), jnp.float32),
            pltpu.VMEM((LANES,), jnp.int32),
            pltpu.VMEM((LANES,), jnp.int32),
            pltpu.VMEM((TOPK, D_IN), jnp.float32),
            pltpu.VMEM((D_IN,), jnp.float32),
            pltpu.VMEM((D_IN,), jnp.float32),
            pltpu.SemaphoreType.DMA,
            pltpu.SemaphoreType.DMA,
            pltpu.SemaphoreType.DMA,
            pltpu.SemaphoreType.DMA,
        ],
    )
    return fn(P, W_dec, b_dec)


@jax.jit
def kernel(x, W_enc, b_enc, W_dec, b_dec):
    sae_in = x - b_dec
    half = N_TOK // 2
    outs = []
    for s in range(2):
        P = _encode(sae_in[s * half:(s + 1) * half], W_enc, b_enc)
        outs.append(_sc_topk_decode(P, W_dec, b_dec))
    return jnp.concatenate(outs, axis=0)


# speedup vs baseline: 1.2408x; 1.0118x over previous
import functools

import jax
import jax.numpy as jnp
from jax import lax
from jax.experimental import pallas as pl
from jax.experimental.pallas import tpu as pltpu
from jax.experimental.pallas import tpu_sc as plsc

N_TOK = 2048
D_IN = 768
HIDDEN = 24576
TOPK = 32

LANES = 16
NWORK = 32
ROWS_PER_W = N_TOK // NWORK
NVEC = HIDDEN // LANES
CHUNK = 256
NCH = HIDDEN // CHUNK
NSUP = NCH // 16


R_B1 = 256
H_B1 = 2048


def _encode_body(x_ref, w_ref, b_ref, p_ref):
    acc = jax.lax.dot_general(
        x_ref[...], w_ref[...], dimension_numbers=(((1,), (1,)), ((), ())),
        preferred_element_type=jnp.float32)
    p_ref[...] = jnp.maximum(acc + b_ref[...], 0.0)


def _encode(x, W_enc, b_enc):
    ntok = x.shape[0]
    grid = (HIDDEN // H_B1, ntok // R_B1)
    return pl.pallas_call(
        _encode_body,
        grid=grid,
        in_specs=[
            pl.BlockSpec((R_B1, D_IN), lambda h, r: (r, 0)),
            pl.BlockSpec((H_B1, D_IN), lambda h, r: (h, 0)),
            pl.BlockSpec((1, H_B1), lambda h, r: (0, h)),
        ],
        out_specs=pl.BlockSpec((R_B1, H_B1), lambda h, r: (r, h)),
        out_shape=jax.ShapeDtypeStruct((ntok, HIDDEN), jnp.float32),
    )(x, W_enc, b_enc.reshape(1, HIDDEN))


_GDN = jax.lax.GatherDimensionNumbers(
    offset_dims=(), collapsed_slice_dims=(0,), start_index_map=(0,))


def _splat(v, k):
    idx = jnp.full((LANES, 1), k, jnp.int32)
    return jax.lax.gather(v, idx, _GDN, (1,),
                          mode=jax.lax.GatherScatterMode.PROMISE_IN_BOUNDS)


def _shuf(v, idx):
    return jax.lax.gather(v, idx.reshape(LANES, 1), _GDN, (1,),
                          mode=jax.lax.GatherScatterMode.PROMISE_IN_BOUNDS)


def _allmax(v):
    lane = jax.lax.iota(jnp.int32, LANES)
    for s in (8, 4, 2, 1):
        v = jnp.maximum(v, _shuf(v, jnp.bitwise_xor(lane, s)))
    return v


def _allmin(v):
    lane = jax.lax.iota(jnp.int32, LANES)
    for s in (8, 4, 2, 1):
        v = jnp.minimum(v, _shuf(v, jnp.bitwise_xor(lane, s)))
    return v


def _scalar0(v):
    return jnp.squeeze(jax.lax.slice(v, (0,), (1,)))


def _tmax(vals):
    vals = list(vals)
    while len(vals) > 1:
        nxt = [jnp.maximum(vals[i], vals[i + 1])
               for i in range(0, len(vals) - 1, 2)]
        if len(vals) % 2:
            nxt.append(vals[-1])
        vals = nxt
    return vals[0]


def _tmin(vals):
    vals = list(vals)
    while len(vals) > 1:
        nxt = [jnp.minimum(vals[i], vals[i + 1])
               for i in range(0, len(vals) - 1, 2)]
        if len(vals) % 2:
            nxt.append(vals[-1])
        vals = nxt
    return vals[0]


def _sc_body(rows_per_w, p_hbm, wdec_hbm, bdec_hbm, out_hbm,
             rowa_v, rowb_v, cm_v, scm_v, sella_v, selhb_v, wrows_v, acc_v,
             bdec_v, sema, semb, wsem, wsem2):
    wid = lax.axis_index("s") * 2 + lax.axis_index("c")
    lane = jnp.arange(LANES, dtype=jnp.int32)
    neg = jnp.float32(float("-inf"))

    pltpu.sync_copy(bdec_hbm, bdec_v)

    z = jnp.full((LANES,), neg, jnp.float32)
    zi = jnp.zeros((LANES,), jnp.int32)
    big = jnp.full((LANES,), 2**30, jnp.int32)

    def process(row_v, t):
        def p1(c2, _):
            for q in range(2):
                c = c2 * 2 + q
                m = _tmax([row_v[pl.ds(c * CHUNK + u * LANES, LANES)]
                           for u in range(CHUNK // LANES)])
                cm_v[pl.ds(c * LANES, LANES)] = m
            return 0
        lax.fori_loop(0, NCH // 2, p1, 0)

        for s in range(NSUP):
            m = _tmax([cm_v[pl.ds((s * 16 + u) * LANES, LANES)]
                       for u in range(16)])
            scm_v[pl.ds(s * LANES, LANES)] = m

        def one_k(k, carry):
            rv0, rv1, ri0, ri1 = carry
            sv = [scm_v[pl.ds(s * LANES, LANES)] for s in range(NSUP)]
            ms = _allmax(_tmax(sv))
            sstar = _scalar0(_allmin(_tmin(
                [jnp.where(sv[s] == ms, s, big) for s in range(NSUP)])))
            gbase = sstar * 16 * LANES
            cv = [cm_v[pl.ds(gbase + u * LANES, LANES)] for u in range(16)]
            cstar = sstar * 16 + _scalar0(_allmin(_tmin(
                [jnp.where(cv[u] == ms, u, big) for u in range(16)])))
            base = cstar * CHUNK

            vs = [row_v[pl.ds(base + u * LANES, LANES)]
                  for u in range(CHUNK // LANES)]
            pos = _allmin(_tmin(
                [jnp.where(vs[u] == ms, u * LANES + lane, big)
                 for u in range(CHUNK // LANES)]))

            masked = []
            for u in range(CHUNK // LANES):
                v = jnp.where((u * LANES + lane) == pos, neg, vs[u])
                row_v[pl.ds(base + u * LANES, LANES)] = v
                masked.append(v)
            nm = _tmax(masked)
            cm_v[pl.ds(cstar * LANES, LANES)] = nm
            rel = cstar - sstar * 16
            sm = _tmax([jnp.where(u == rel, nm, cv[u]) for u in range(16)])
            scm_v[pl.ds(sstar * LANES, LANES)] = sm

            mi = base + pos
            rv0 = jnp.where(lane == k, ms, rv0)
            ri0 = jnp.where(lane == k, mi, ri0)
            rv1 = jnp.where(lane == k - 16, ms, rv1)
            ri1 = jnp.where(lane == k - 16, mi, ri1)
            return rv0, rv1, ri0, ri1

        rv0, rv1, ri0, ri1 = lax.fori_loop(0, TOPK, one_k, (z, z, zi, zi))
        return rv0, rv1, ri0, ri1

    def decode_issue(ri0, ri1):
        sella_v[...] = ri0
        selhb_v[...] = ri1
        cp1 = pltpu.async_copy(wdec_hbm.at[sella_v],
                               wrows_v.at[pl.ds(0, LANES)], wsem)
        cp2 = pltpu.async_copy(wdec_hbm.at[selhb_v],
                               wrows_v.at[pl.ds(LANES, LANES)], wsem2)
        return cp1, cp2

    def decode_finish(t, rv0, rv1, cp1, cp2):
        ws_lo = [_splat(rv0, k) for k in range(LANES)]
        ws_hi = [_splat(rv1, k) for k in range(LANES)]
        cp1.wait()

        def dj1(j, _):
            sl = pl.ds(j * LANES, LANES)
            a = bdec_v[sl]
            for k in range(LANES):
                a = a + ws_lo[k] * wrows_v[k, sl]
            acc_v[sl] = a
            return 0
        lax.fori_loop(0, D_IN // LANES, dj1, 0)
        cp2.wait()

        def dj2(j, _):
            sl = pl.ds(j * LANES, LANES)
            a = acc_v[sl]
            for k in range(LANES):
                a = a + ws_hi[k] * wrows_v[LANES + k, sl]
            acc_v[sl] = a
            return 0
        lax.fori_loop(0, D_IN // LANES, dj2, 0)
        pltpu.sync_copy(acc_v, out_hbm.at[t])

    t0 = wid * rows_per_w
    pltpu.async_copy(p_hbm.at[t0], rowa_v, sema)

    def two_rows(ii, _):
        ta = t0 + 2 * ii
        pltpu.make_async_copy(p_hbm.at[ta], rowa_v, sema).wait()
        pltpu.async_copy(p_hbm.at[ta + 1], rowb_v, semb)
        rv0, rv1, ri0, ri1 = process(rowa_v, ta)
        cpa1, cpa2 = decode_issue(ri0, ri1)
        pltpu.make_async_copy(p_hbm.at[ta + 1], rowb_v, semb).wait()

        @pl.when(ii < rows_per_w // 2 - 1)
        def _():
            pltpu.async_copy(p_hbm.at[ta + 2], rowa_v, sema)
        rv0b, rv1b, ri0b, ri1b = process(rowb_v, ta + 1)
        decode_finish(ta, rv0, rv1, cpa1, cpa2)
        cpb1, cpb2 = decode_issue(ri0b, ri1b)
        decode_finish(ta + 1, rv0b, rv1b, cpb1, cpb2)
        return 0

    lax.fori_loop(0, rows_per_w // 2, two_rows, 0)


def _sc_topk_decode(P, W_dec, b_dec):
    ntok = P.shape[0]
    mesh = plsc.VectorSubcoreMesh(core_axis_name="c", subcore_axis_name="s")
    fn = pl.kernel(
        functools.partial(_sc_body, ntok // NWORK), mesh=mesh,
        out_type=jax.ShapeDtypeStruct((ntok, D_IN), jnp.float32),
        scratch_types=[
            pltpu.VMEM((HIDDEN,), jnp.float32),
            pltpu.VMEM((HIDDEN,), jnp.float32),
            pltpu.VMEM((NCH * LANES,), jnp.float32),
            pltpu.VMEM((NSUP * LANES,), jnp.float32),
            pltpu.VMEM((LANES,), jnp.int32),
            pltpu.VMEM((LANES,), jnp.int32),
            pltpu.VMEM((TOPK, D_IN), jnp.float32),
            pltpu.VMEM((D_IN,), jnp.float32),
            pltpu.VMEM((D_IN,), jnp.float32),
            pltpu.SemaphoreType.DMA,
            pltpu.SemaphoreType.DMA,
            pltpu.SemaphoreType.DMA,
            pltpu.SemaphoreType.DMA,
        ],
    )
    return fn(P, W_dec, b_dec)


@jax.jit
def kernel(x, W_enc, b_enc, W_dec, b_dec):
    sae_in = x - b_dec
    half = N_TOK // 4
    outs = []
    for s in range(4):
        P = _encode(sae_in[s * half:(s + 1) * half], W_enc, b_enc)
        outs.append(_sc_topk_decode(P, W_dec, b_dec))
    return jnp.concatenate(outs, axis=0)


# speedup vs baseline: 1.2410x; 1.0002x over previous
import functools

import jax
import jax.numpy as jnp
from jax import lax
from jax.experimental import pallas as pl
from jax.experimental.pallas import tpu as pltpu
from jax.experimental.pallas import tpu_sc as plsc

N_TOK = 2048
D_IN = 768
HIDDEN = 24576
TOPK = 32

LANES = 16
NWORK = 32
CHUNK = 256
NCH = HIDDEN // CHUNK
NSUP = NCH // 16


R_B1 = 256
H_B1 = 2048


def _encode_body(x_ref, w_ref, b_ref, p_ref):
    acc = jax.lax.dot_general(
        x_ref[...], w_ref[...], dimension_numbers=(((1,), (1,)), ((), ())),
        preferred_element_type=jnp.float32)
    p_ref[...] = jnp.maximum(acc + b_ref[...], 0.0)


def _encode(x, W_enc, b_enc):
    ntok = x.shape[0]
    grid = (HIDDEN // H_B1, ntok // R_B1)
    return pl.pallas_call(
        _encode_body,
        grid=grid,
        in_specs=[
            pl.BlockSpec((R_B1, D_IN), lambda h, r: (r, 0)),
            pl.BlockSpec((H_B1, D_IN), lambda h, r: (h, 0)),
            pl.BlockSpec((1, H_B1), lambda h, r: (0, h)),
        ],
        out_specs=pl.BlockSpec((R_B1, H_B1), lambda h, r: (r, h)),
        out_shape=jax.ShapeDtypeStruct((ntok, HIDDEN), jnp.float32),
    )(x, W_enc, b_enc.reshape(1, HIDDEN))


_GDN = jax.lax.GatherDimensionNumbers(
    offset_dims=(), collapsed_slice_dims=(0,), start_index_map=(0,))


def _splat(v, k):
    idx = jnp.full((LANES, 1), k, jnp.int32)
    return jax.lax.gather(v, idx, _GDN, (1,),
                          mode=jax.lax.GatherScatterMode.PROMISE_IN_BOUNDS)


def _shuf(v, idx):
    return jax.lax.gather(v, idx.reshape(LANES, 1), _GDN, (1,),
                          mode=jax.lax.GatherScatterMode.PROMISE_IN_BOUNDS)


def _allmax(v):
    lane = jax.lax.iota(jnp.int32, LANES)
    for s in (8, 4, 2, 1):
        v = jnp.maximum(v, _shuf(v, jnp.bitwise_xor(lane, s)))
    return v


def _allmin(v):
    lane = jax.lax.iota(jnp.int32, LANES)
    for s in (8, 4, 2, 1):
        v = jnp.minimum(v, _shuf(v, jnp.bitwise_xor(lane, s)))
    return v


def _scalar0(v):
    return jnp.squeeze(jax.lax.slice(v, (0,), (1,)))


def _tmax(vals):
    vals = list(vals)
    while len(vals) > 1:
        nxt = [jnp.maximum(vals[i], vals[i + 1])
               for i in range(0, len(vals) - 1, 2)]
        if len(vals) % 2:
            nxt.append(vals[-1])
        vals = nxt
    return vals[0]


def _tmin(vals):
    vals = list(vals)
    while len(vals) > 1:
        nxt = [jnp.minimum(vals[i], vals[i + 1])
               for i in range(0, len(vals) - 1, 2)]
        if len(vals) % 2:
            nxt.append(vals[-1])
        vals = nxt
    return vals[0]


def _sc_body(rows_per_w, p_hbm, wdec_hbm, bdec_hbm, out_hbm,
             rowa_v, rowb_v, cm_v, scm_v, sella_v, selhb_v, wrows_v, acc_v,
             bdec_v, sema, semb, wsem, wsem2):
    wid = lax.axis_index("s") * 2 + lax.axis_index("c")
    lane = jnp.arange(LANES, dtype=jnp.int32)
    neg = jnp.float32(float("-inf"))

    pltpu.sync_copy(bdec_hbm, bdec_v)

    z = jnp.full((LANES,), neg, jnp.float32)
    zi = jnp.zeros((LANES,), jnp.int32)
    big = jnp.full((LANES,), 2**30, jnp.int32)

    def process(row_v, t):
        def p1(c2, _):
            for q in range(2):
                c = c2 * 2 + q
                m = _tmax([row_v[pl.ds(c * CHUNK + u * LANES, LANES)]
                           for u in range(CHUNK // LANES)])
                cm_v[pl.ds(c * LANES, LANES)] = m
            return 0
        lax.fori_loop(0, NCH // 2, p1, 0)

        for s in range(NSUP):
            m = _tmax([cm_v[pl.ds((s * 16 + u) * LANES, LANES)]
                       for u in range(16)])
            scm_v[pl.ds(s * LANES, LANES)] = m

        def one_k(k, carry):
            rv0, rv1, ri0, ri1 = carry
            sv = [scm_v[pl.ds(s * LANES, LANES)] for s in range(NSUP)]
            ms = _allmax(_tmax(sv))
            sstar = _scalar0(_allmin(_tmin(
                [jnp.where(sv[s] == ms, s, big) for s in range(NSUP)])))
            gbase = sstar * 16 * LANES
            cv = [cm_v[pl.ds(gbase + u * LANES, LANES)] for u in range(16)]
            cstar = sstar * 16 + _scalar0(_allmin(_tmin(
                [jnp.where(cv[u] == ms, u, big) for u in range(16)])))
            base = cstar * CHUNK

            vs = [row_v[pl.ds(base + u * LANES, LANES)]
                  for u in range(CHUNK // LANES)]
            pos = _allmin(_tmin(
                [jnp.where(vs[u] == ms, u * LANES + lane, big)
                 for u in range(CHUNK // LANES)]))

            masked = []
            for u in range(CHUNK // LANES):
                v = jnp.where((u * LANES + lane) == pos, neg, vs[u])
                row_v[pl.ds(base + u * LANES, LANES)] = v
                masked.append(v)
            nm = _tmax(masked)
            cm_v[pl.ds(cstar * LANES, LANES)] = nm
            rel = cstar - sstar * 16
            sm = _tmax([jnp.where(u == rel, nm, cv[u]) for u in range(16)])
            scm_v[pl.ds(sstar * LANES, LANES)] = sm

            mi = base + pos
            rv0 = jnp.where(lane == k, ms, rv0)
            ri0 = jnp.where(lane == k, mi, ri0)
            rv1 = jnp.where(lane == k - 16, ms, rv1)
            ri1 = jnp.where(lane == k - 16, mi, ri1)
            return rv0, rv1, ri0, ri1

        rv0, rv1, ri0, ri1 = lax.fori_loop(0, TOPK, one_k, (z, z, zi, zi))
        return rv0, rv1, ri0, ri1

    def decode_issue(ri0, ri1):
        sella_v[...] = ri0
        selhb_v[...] = ri1
        cp1 = pltpu.async_copy(wdec_hbm.at[sella_v],
                               wrows_v.at[pl.ds(0, LANES)], wsem)
        cp2 = pltpu.async_copy(wdec_hbm.at[selhb_v],
                               wrows_v.at[pl.ds(LANES, LANES)], wsem2)
        return cp1, cp2

    def decode_finish(t, rv0, rv1, cp1, cp2):
        ws_lo = [_splat(rv0, k) for k in range(LANES)]
        ws_hi = [_splat(rv1, k) for k in range(LANES)]
        cp1.wait()

        def dj1(j, _):
            sl = pl.ds(j * LANES, LANES)
            a = bdec_v[sl]
            for k in range(LANES):
                a = a + ws_lo[k] * wrows_v[k, sl]
            acc_v[sl] = a
            return 0
        lax.fori_loop(0, D_IN // LANES, dj1, 0)
        cp2.wait()

        def dj2(j, _):
            sl = pl.ds(j * LANES, LANES)
            a = acc_v[sl]
            for k in range(LANES):
                a = a + ws_hi[k] * wrows_v[LANES + k, sl]
            acc_v[sl] = a
            return 0
        lax.fori_loop(0, D_IN // LANES, dj2, 0)
        pltpu.sync_copy(acc_v, out_hbm.at[t])

    t0 = wid * rows_per_w
    pltpu.async_copy(p_hbm.at[t0], rowa_v, sema)

    def two_rows(ii, _):
        ta = t0 + 2 * ii
        pltpu.make_async_copy(p_hbm.at[ta], rowa_v, sema).wait()
        pltpu.async_copy(p_hbm.at[ta + 1], rowb_v, semb)
        rv0, rv1, ri0, ri1 = process(rowa_v, ta)
        cpa1, cpa2 = decode_issue(ri0, ri1)
        pltpu.make_async_copy(p_hbm.at[ta + 1], rowb_v, semb).wait()

        @pl.when(ii < rows_per_w // 2 - 1)
        def _():
            pltpu.async_copy(p_hbm.at[ta + 2], rowa_v, sema)
        rv0b, rv1b, ri0b, ri1b = process(rowb_v, ta + 1)
        decode_finish(ta, rv0, rv1, cpa1, cpa2)
        cpb1, cpb2 = decode_issue(ri0b, ri1b)
        decode_finish(ta + 1, rv0b, rv1b, cpb1, cpb2)
        return 0

    lax.fori_loop(0, rows_per_w // 2, two_rows, 0)


def _sc_topk_decode(P, W_dec, b_dec):
    ntok = P.shape[0]
    mesh = plsc.VectorSubcoreMesh(core_axis_name="c", subcore_axis_name="s")
    fn = pl.kernel(
        functools.partial(_sc_body, ntok // NWORK), mesh=mesh,
        out_type=jax.ShapeDtypeStruct((ntok, D_IN), jnp.float32),
        scratch_types=[
            pltpu.VMEM((HIDDEN,), jnp.float32),
            pltpu.VMEM((HIDDEN,), jnp.float32),
            pltpu.VMEM((NCH * LANES,), jnp.float32),
            pltpu.VMEM((NSUP * LANES,), jnp.float32),
            pltpu.VMEM((LANES,), jnp.int32),
            pltpu.VMEM((LANES,), jnp.int32),
            pltpu.VMEM((TOPK, D_IN), jnp.float32),
            pltpu.VMEM((D_IN,), jnp.float32),
            pltpu.VMEM((D_IN,), jnp.float32),
            pltpu.SemaphoreType.DMA,
            pltpu.SemaphoreType.DMA,
            pltpu.SemaphoreType.DMA,
            pltpu.SemaphoreType.DMA,
        ],
    )
    return fn(P, W_dec, b_dec)


@jax.jit
def kernel(x, W_enc, b_enc, W_dec, b_dec):
    sae_in = x - b_dec
    half = N_TOK // 4
    outs = []
    for s in range(4):
        P = _encode(sae_in[s * half:(s + 1) * half], W_enc, b_enc)
        outs.append(_sc_topk_decode(P, W_dec, b_dec))
    return jnp.concatenate(outs, axis=0)
